# baseline (speedup 1.0000x reference)
import functools

import jax
import jax.numpy as jnp
import numpy as np
from jax.experimental import pallas as pl
import jax.experimental.pallas.tpu as pltpu
from jax.experimental.pallas import tpu_sc as plsc

FRAG = 64
HALF = FRAG // 2
G = 6
P = 24
PPAD = 32
GPAD = 8
KQ = 5
TOP_U = 16
W_MEAN, W_MAX, W_TOPK, W_STD = 1.0, 0.6, 0.4, 0.2
ALPHA = 0.15
TAU = 8.0
NEG = -1e30


def _stats_kernel(q_ref, k_ref, u_ref, st_ref, *, T, d):
    kh = k_ref[0, 0]
    qh = q_ref[0, 0]

    r32 = jax.lax.broadcasted_iota(jnp.int32, (PPAD, 1), 0)
    step = float(T - 1) / float(P - 1)
    idxp = jnp.where(r32 < P,
                     jnp.round(r32.astype(jnp.float32) * step).astype(jnp.int32),
                     -1)
    colT = jax.lax.broadcasted_iota(jnp.int32, (PPAD, T), 1)
    onehot_p = (colT == idxp).astype(jnp.float32)
    qp = jax.lax.dot_general(onehot_p, qh, (((1,), (0,)), ((), ())),
                             preferred_element_type=jnp.float32,
                             precision=jax.lax.Precision.HIGHEST)
    qnorm = jnp.sqrt(jnp.sum(qp * qp, axis=1, keepdims=True))
    qp = qp / jnp.maximum(qnorm, 1e-6)

    knorm = jnp.sqrt(jnp.sum(kh * kh, axis=1, keepdims=True))
    kbar = kh / jnp.maximum(knorm, 1e-6)

    st = jax.lax.dot_general(qp.astype(jnp.bfloat16), kbar.astype(jnp.bfloat16),
                             (((1,), (1,)), ((), ())),
                             preferred_element_type=jnp.float32)
    st = jnp.maximum(st, 0.0)

    mean = jnp.sum(st, axis=0, keepdims=True) / P
    mx = jnp.max(st, axis=0, keepdims=True)
    ssq = jnp.sum(st * st, axis=0, keepdims=True)
    std = jnp.sqrt(jnp.maximum(ssq / P - mean * mean, 0.0))

    sub = jax.lax.broadcasted_iota(jnp.int32, (PPAD, T), 0)
    cur = st
    acc = jnp.zeros((1, T), jnp.float32)
    for _ in range(KQ):
        m5 = jnp.max(cur, axis=0, keepdims=True)
        acc = acc + m5
        first = jnp.min(jnp.where(cur >= m5, sub, PPAD), axis=0, keepdims=True)
        cur = jnp.where(sub == first, -1.0, cur)
    topk_mean = acc / KQ

    u = W_MEAN * mean + W_MAX * mx + W_TOPK * topk_mean + W_STD * std
    u_ref[0, 0] = u
    st_ref[0, 0] = st


def _router_sc_body(u_hbm, s_hbm, out_hbm, u_v, s_v, cho_v, *, T, H):
    c = jax.lax.axis_index("c")
    s = jax.lax.axis_index("s")
    b = s // H
    h = s - b * H

    @pl.when(c == 0)
    def _work():
        pltpu.sync_copy(u_hbm.at[b, h, 0], u_v)
        pltpu.sync_copy(s_hbm.at[b, h, pl.ds(0, P)], s_v)
        iota16 = jax.lax.iota(jnp.int32, 16)
        neg16 = jnp.full((16,), -3e38, jnp.float32)
        cand = jnp.zeros((16,), jnp.int32)
        nch = T // 16

        for r in range(TOP_U):
            def scan_body(ci, carry):
                bv, bi = carry
                chunk = u_v[pl.ds(ci * 16, 16)]
                m = jnp.max(chunk)
                fl = jnp.max(plsc.all_reduce_ffs(chunk == m))
                idx = ci * 16 + fl
                better = m > bv
                return (jnp.where(better, m, bv),
                        jnp.where(better, idx, bi))
            bv, bi = jax.lax.fori_loop(
                0, nch, scan_body,
                (jnp.float32(-3e38), jnp.int32(0)))
            c0 = bi // 16
            l0 = bi - c0 * 16
            ch = u_v[pl.ds(c0 * 16, 16)]
            u_v[pl.ds(c0 * 16, 16)] = jnp.where(iota16 == l0, neg16, ch)
            cand = jnp.where(iota16 == r, bi, cand)

        ssub = []
        for p in range(P):
            ssub.append(plsc.load_gather(
                s_v, [jnp.full((16,), p, jnp.int32), cand]))

        msub = [jnp.float32(0.0)] * P
        blocked = jnp.zeros((16,), jnp.bool_)
        chosen = jnp.zeros((16,), jnp.int32)
        for r in range(G):
            gains = jnp.zeros((16,), jnp.float32)
            for p in range(P):
                gains = gains + jnp.maximum(ssub[p] - msub[p], 0.0)
            gains = jnp.where(blocked, jnp.float32(-1e9), gains)
            mg = jnp.max(gains)
            j = jnp.max(plsc.all_reduce_ffs(gains == mg))
            ohj = iota16 == j
            blocked = blocked | ohj
            cj = jnp.sum(jnp.where(ohj, cand, 0))
            chosen = jnp.where(iota16 == r, cj, chosen)
            for p in range(P):
                sp = jnp.sum(jnp.where(ohj, ssub[p], jnp.float32(0.0)))
                msub[p] = jnp.maximum(msub[p], sp)
        cho_v[...] = chosen
        pltpu.sync_copy(cho_v, out_hbm.at[b, h])


def _attn_kernel(chosen_ref, q_ref, k_ref, v_ref, o_ref, *, T, d, C, HALO):
    bb = pl.program_id(0)
    h = pl.program_id(1)
    b = pl.program_id(2)
    s0 = b * C
    halo_start = jnp.clip(s0 - HALF, 0, T - HALO)

    qb = q_ref[0, 0]
    k_halo = k_ref[0, 0, pl.ds(halo_start, HALO), :]
    v_halo = v_ref[0, 0, pl.ds(halo_start, HALO), :]

    inv_sqrt_d = 1.0 / float(np.sqrt(d))
    qb16 = qb.astype(jnp.bfloat16)
    sl = jax.lax.dot_general(qb16, k_halo.astype(jnp.bfloat16),
                             (((1,), (1,)), ((), ())),
                             preferred_element_type=jnp.float32) * inv_sqrt_d
    t_abs = s0 + jax.lax.broadcasted_iota(jnp.int32, (C, HALO), 0)
    j_abs = halo_start + jax.lax.broadcasted_iota(jnp.int32, (C, HALO), 1)
    start_t = jnp.clip(t_abs - HALF, 0, T - FRAG)
    valid = (j_abs >= start_t) & (j_abs < start_t + FRAG)
    prior = jnp.exp(jnp.abs(j_abs - t_abs).astype(jnp.float32) * (-1.0 / TAU))
    sl = jnp.where(valid, sl + ALPHA * prior, NEG)

    krows = []
    vrows = []
    cvals = []
    for i in range(G):
        ci = chosen_ref[bb, h, i]
        krows.append(k_ref[0, 0, pl.ds(ci, 1), :])
        vrows.append(v_ref[0, 0, pl.ds(ci, 1), :])
        cvals.append(ci.reshape(1, 1))
    zpad = jnp.zeros((GPAD - G, d), jnp.float32)
    kg = jnp.concatenate(krows + [zpad], axis=0)
    vg = jnp.concatenate(vrows + [zpad], axis=0)
    crow = jnp.concatenate(
        cvals + [jnp.zeros((1, GPAD - G), jnp.int32)], axis=1)

    sg = jax.lax.dot_general(qb16, kg.astype(jnp.bfloat16),
                             (((1,), (1,)), ((), ())),
                             preferred_element_type=jnp.float32) * inv_sqrt_d
    tq = s0 + jax.lax.broadcasted_iota(jnp.int32, (C, GPAD), 0)
    lane8 = jax.lax.broadcasted_iota(jnp.int32, (C, GPAD), 1)
    priorg = jnp.exp(jnp.abs(crow - tq).astype(jnp.float32) * (-1.0 / TAU))
    sg = jnp.where(lane8 < G, sg + ALPHA * priorg, NEG)

    mm = jnp.maximum(jnp.max(sl, axis=1, keepdims=True),
                     jnp.max(sg, axis=1, keepdims=True))
    pl_ = jnp.exp(sl - mm)
    pg = jnp.exp(sg - mm)
    denom = jnp.sum(pl_, axis=1, keepdims=True) + jnp.sum(pg, axis=1, keepdims=True)
    ctx = (jnp.dot(pl_.astype(jnp.bfloat16), v_halo.astype(jnp.bfloat16),
                   preferred_element_type=jnp.float32)
           + jnp.dot(pg.astype(jnp.bfloat16), vg.astype(jnp.bfloat16),
                     preferred_element_type=jnp.float32)) / denom
    o_ref[0, 0] = ctx


@jax.jit
def kernel(q, k, v):
    B, H, T, d = q.shape

    u4, st4 = pl.pallas_call(
        functools.partial(_stats_kernel, T=T, d=d),
        grid=(B, H),
        in_specs=[
            pl.BlockSpec((1, 1, T, d), lambda b, h: (b, h, 0, 0)),
            pl.BlockSpec((1, 1, T, d), lambda b, h: (b, h, 0, 0)),
        ],
        out_specs=[
            pl.BlockSpec((1, 1, 1, T), lambda b, h: (b, h, 0, 0)),
            pl.BlockSpec((1, 1, PPAD, T), lambda b, h: (b, h, 0, 0)),
        ],
        out_shape=[
            jax.ShapeDtypeStruct((B, H, 1, T), jnp.float32),
            jax.ShapeDtypeStruct((B, H, PPAD, T), jnp.float32),
        ],
        compiler_params=pltpu.CompilerParams(
            dimension_semantics=("parallel", "parallel")),
    )(q, k)

    mesh = plsc.VectorSubcoreMesh(core_axis_name="c", subcore_axis_name="s")
    chosen = pl.kernel(
        functools.partial(_router_sc_body, T=T, H=H),
        out_type=jax.ShapeDtypeStruct((B, H, 16), jnp.int32),
        mesh=mesh,
        scratch_types=[
            pltpu.VMEM((T,), jnp.float32),
            pltpu.VMEM((P, T), jnp.float32),
            pltpu.VMEM((16,), jnp.int32),
        ],
        compiler_params=pltpu.CompilerParams(needs_layout_passes=False),
    )(u4, st4)

    C = 256
    HALO = C + FRAG
    NB = T // C
    grid_spec = pltpu.PrefetchScalarGridSpec(
        num_scalar_prefetch=1,
        grid=(B, H, NB),
        in_specs=[
            pl.BlockSpec((1, 1, C, d), lambda bb, h, b, ch: (bb, h, b, 0)),
            pl.BlockSpec((1, 1, T, d), lambda bb, h, b, ch: (bb, h, 0, 0)),
            pl.BlockSpec((1, 1, T, d), lambda bb, h, b, ch: (bb, h, 0, 0)),
        ],
        out_specs=pl.BlockSpec((1, 1, C, d), lambda bb, h, b, ch: (bb, h, b, 0)),
    )
    return pl.pallas_call(
        functools.partial(_attn_kernel, T=T, d=d, C=C, HALO=HALO),
        grid_spec=grid_spec,
        out_shape=jax.ShapeDtypeStruct((B, H, T, d), jnp.float32),
        compiler_params=pltpu.CompilerParams(
            dimension_semantics=("parallel", "parallel", "arbitrary")),
    )(chosen, q, k, v)


# speedup vs baseline: 1.0033x; 1.0033x over previous
import functools

import jax
import jax.numpy as jnp
import numpy as np
from jax.experimental import pallas as pl
import jax.experimental.pallas.tpu as pltpu
from jax.experimental.pallas import tpu_sc as plsc

FRAG = 64
HALF = FRAG // 2
G = 6
P = 24
PPAD = 32
GPAD = 8
KQ = 5
TOP_U = 16
W_MEAN, W_MAX, W_TOPK, W_STD = 1.0, 0.6, 0.4, 0.2
ALPHA = 0.15
TAU = 8.0
NEG = -1e30


def _stats_kernel(q_ref, k_ref, u_ref, st_ref, *, T, d):
    kh = k_ref[0, 0]
    qh = q_ref[0, 0]

    r32 = jax.lax.broadcasted_iota(jnp.int32, (PPAD, 1), 0)
    step = float(T - 1) / float(P - 1)
    idxp = jnp.where(r32 < P,
                     jnp.round(r32.astype(jnp.float32) * step).astype(jnp.int32),
                     -1)
    colT = jax.lax.broadcasted_iota(jnp.int32, (PPAD, T), 1)
    onehot_p = (colT == idxp).astype(jnp.float32)
    qp = jax.lax.dot_general(onehot_p, qh, (((1,), (0,)), ((), ())),
                             preferred_element_type=jnp.float32,
                             precision=jax.lax.Precision.HIGHEST)
    qnorm = jnp.sqrt(jnp.sum(qp * qp, axis=1, keepdims=True))
    qp = qp / jnp.maximum(qnorm, 1e-6)

    knorm = jnp.sqrt(jnp.sum(kh * kh, axis=1, keepdims=True))
    kbar = kh / jnp.maximum(knorm, 1e-6)

    st = jax.lax.dot_general(qp.astype(jnp.bfloat16), kbar.astype(jnp.bfloat16),
                             (((1,), (1,)), ((), ())),
                             preferred_element_type=jnp.float32)
    st = jnp.maximum(st, 0.0)

    mean = jnp.sum(st, axis=0, keepdims=True) / P
    mx = jnp.max(st, axis=0, keepdims=True)
    ssq = jnp.sum(st * st, axis=0, keepdims=True)
    std = jnp.sqrt(jnp.maximum(ssq / P - mean * mean, 0.0))

    sub = jax.lax.broadcasted_iota(jnp.int32, (PPAD, T), 0)
    cur = st
    acc = jnp.zeros((1, T), jnp.float32)
    for _ in range(KQ):
        m5 = jnp.max(cur, axis=0, keepdims=True)
        acc = acc + m5
        first = jnp.min(jnp.where(cur >= m5, sub, PPAD), axis=0, keepdims=True)
        cur = jnp.where(sub == first, -1.0, cur)
    topk_mean = acc / KQ

    u = W_MEAN * mean + W_MAX * mx + W_TOPK * topk_mean + W_STD * std
    u_ref[0, 0] = u
    st_ref[0, 0] = st[:P]


def _router_sc_body(u_hbm, s_hbm, out_hbm, u_v, s_v, cho_v, *, T, H):
    c = jax.lax.axis_index("c")
    s = jax.lax.axis_index("s")
    b = s // H
    h = s - b * H

    @pl.when(c == 0)
    def _work():
        pltpu.sync_copy(u_hbm.at[b, h, 0], u_v)
        pltpu.sync_copy(s_hbm.at[b, h], s_v)
        iota16 = jax.lax.iota(jnp.int32, 16)
        neg16 = jnp.full((16,), -3e38, jnp.float32)
        cand = jnp.zeros((16,), jnp.int32)
        nch = T // 16

        for r in range(TOP_U):
            def scan_body(ci, carry):
                bv, bi = carry
                chunk = u_v[pl.ds(ci * 16, 16)]
                m = jnp.max(chunk)
                fl = jnp.max(plsc.all_reduce_ffs(chunk == m))
                idx = ci * 16 + fl
                better = m > bv
                return (jnp.where(better, m, bv),
                        jnp.where(better, idx, bi))
            bv, bi = jax.lax.fori_loop(
                0, nch, scan_body,
                (jnp.float32(-3e38), jnp.int32(0)))
            c0 = bi // 16
            l0 = bi - c0 * 16
            ch = u_v[pl.ds(c0 * 16, 16)]
            u_v[pl.ds(c0 * 16, 16)] = jnp.where(iota16 == l0, neg16, ch)
            cand = jnp.where(iota16 == r, bi, cand)

        ssub = []
        for p in range(P):
            ssub.append(plsc.load_gather(
                s_v, [jnp.full((16,), p, jnp.int32), cand]))

        msub = [jnp.float32(0.0)] * P
        blocked = jnp.zeros((16,), jnp.bool_)
        chosen = jnp.zeros((16,), jnp.int32)
        for r in range(G):
            gains = jnp.zeros((16,), jnp.float32)
            for p in range(P):
                gains = gains + jnp.maximum(ssub[p] - msub[p], 0.0)
            gains = jnp.where(blocked, jnp.float32(-1e9), gains)
            mg = jnp.max(gains)
            j = jnp.max(plsc.all_reduce_ffs(gains == mg))
            ohj = iota16 == j
            blocked = blocked | ohj
            cj = jnp.sum(jnp.where(ohj, cand, 0))
            chosen = jnp.where(iota16 == r, cj, chosen)
            for p in range(P):
                sp = jnp.sum(jnp.where(ohj, ssub[p], jnp.float32(0.0)))
                msub[p] = jnp.maximum(msub[p], sp)
        cho_v[...] = chosen
        pltpu.sync_copy(cho_v, out_hbm.at[b, h])


def _attn_kernel(chosen_ref, q_ref, k_ref, v_ref, o_ref, *, T, d, C, HALO):
    bb = pl.program_id(0)
    h = pl.program_id(1)
    b = pl.program_id(2)
    s0 = b * C
    halo_start = jnp.clip(s0 - HALF, 0, T - HALO)

    qb = q_ref[0, 0]
    k_halo = k_ref[0, 0, pl.ds(halo_start, HALO), :]
    v_halo = v_ref[0, 0, pl.ds(halo_start, HALO), :]

    inv_sqrt_d = 1.0 / float(np.sqrt(d))
    qb16 = qb.astype(jnp.bfloat16)
    sl = jax.lax.dot_general(qb16, k_halo.astype(jnp.bfloat16),
                             (((1,), (1,)), ((), ())),
                             preferred_element_type=jnp.float32) * inv_sqrt_d
    t_abs = s0 + jax.lax.broadcasted_iota(jnp.int32, (C, HALO), 0)
    j_abs = halo_start + jax.lax.broadcasted_iota(jnp.int32, (C, HALO), 1)
    start_t = jnp.clip(t_abs - HALF, 0, T - FRAG)
    valid = (j_abs >= start_t) & (j_abs < start_t + FRAG)
    prior = jnp.exp(jnp.abs(j_abs - t_abs).astype(jnp.float32) * (-1.0 / TAU))
    sl = jnp.where(valid, sl + ALPHA * prior, NEG)

    krows = []
    vrows = []
    cvals = []
    for i in range(G):
        ci = chosen_ref[bb, h, i]
        krows.append(k_ref[0, 0, pl.ds(ci, 1), :])
        vrows.append(v_ref[0, 0, pl.ds(ci, 1), :])
        cvals.append(ci.reshape(1, 1))
    zpad = jnp.zeros((GPAD - G, d), jnp.float32)
    kg = jnp.concatenate(krows + [zpad], axis=0)
    vg = jnp.concatenate(vrows + [zpad], axis=0)
    crow = jnp.concatenate(
        cvals + [jnp.zeros((1, GPAD - G), jnp.int32)], axis=1)

    sg = jax.lax.dot_general(qb16, kg.astype(jnp.bfloat16),
                             (((1,), (1,)), ((), ())),
                             preferred_element_type=jnp.float32) * inv_sqrt_d
    tq = s0 + jax.lax.broadcasted_iota(jnp.int32, (C, GPAD), 0)
    lane8 = jax.lax.broadcasted_iota(jnp.int32, (C, GPAD), 1)
    priorg = jnp.exp(jnp.abs(crow - tq).astype(jnp.float32) * (-1.0 / TAU))
    sg = jnp.where(lane8 < G, sg + ALPHA * priorg, NEG)

    mm = jnp.maximum(jnp.max(sl, axis=1, keepdims=True),
                     jnp.max(sg, axis=1, keepdims=True))
    pl_ = jnp.exp(sl - mm)
    pg = jnp.exp(sg - mm)
    denom = jnp.sum(pl_, axis=1, keepdims=True) + jnp.sum(pg, axis=1, keepdims=True)
    ctx = (jnp.dot(pl_.astype(jnp.bfloat16), v_halo.astype(jnp.bfloat16),
                   preferred_element_type=jnp.float32)
           + jnp.dot(pg.astype(jnp.bfloat16), vg.astype(jnp.bfloat16),
                     preferred_element_type=jnp.float32)) / denom
    o_ref[0, 0] = ctx


@jax.jit
def kernel(q, k, v):
    B, H, T, d = q.shape

    u4, st4 = pl.pallas_call(
        functools.partial(_stats_kernel, T=T, d=d),
        grid=(B, H),
        in_specs=[
            pl.BlockSpec((1, 1, T, d), lambda b, h: (b, h, 0, 0)),
            pl.BlockSpec((1, 1, T, d), lambda b, h: (b, h, 0, 0)),
        ],
        out_specs=[
            pl.BlockSpec((1, 1, 1, T), lambda b, h: (b, h, 0, 0)),
            pl.BlockSpec((1, 1, P, T), lambda b, h: (b, h, 0, 0)),
        ],
        out_shape=[
            jax.ShapeDtypeStruct((B, H, 1, T), jnp.float32),
            jax.ShapeDtypeStruct((B, H, P, T), jnp.float32),
        ],
        compiler_params=pltpu.CompilerParams(
            dimension_semantics=("parallel", "parallel")),
    )(q, k)

    mesh = plsc.VectorSubcoreMesh(core_axis_name="c", subcore_axis_name="s")
    chosen = pl.kernel(
        functools.partial(_router_sc_body, T=T, H=H),
        out_type=jax.ShapeDtypeStruct((B, H, 16), jnp.int32),
        mesh=mesh,
        scratch_types=[
            pltpu.VMEM((T,), jnp.float32),
            pltpu.VMEM((P, T), jnp.float32),
            pltpu.VMEM((16,), jnp.int32),
        ],
        compiler_params=pltpu.CompilerParams(needs_layout_passes=False),
    )(u4, st4)

    C = 256
    HALO = C + FRAG
    NB = T // C
    grid_spec = pltpu.PrefetchScalarGridSpec(
        num_scalar_prefetch=1,
        grid=(B, H, NB),
        in_specs=[
            pl.BlockSpec((1, 1, C, d), lambda bb, h, b, ch: (bb, h, b, 0)),
            pl.BlockSpec((1, 1, T, d), lambda bb, h, b, ch: (bb, h, 0, 0)),
            pl.BlockSpec((1, 1, T, d), lambda bb, h, b, ch: (bb, h, 0, 0)),
        ],
        out_specs=pl.BlockSpec((1, 1, C, d), lambda bb, h, b, ch: (bb, h, b, 0)),
    )
    return pl.pallas_call(
        functools.partial(_attn_kernel, T=T, d=d, C=C, HALO=HALO),
        grid_spec=grid_spec,
        out_shape=jax.ShapeDtypeStruct((B, H, T, d), jnp.float32),
        compiler_params=pltpu.CompilerParams(
            dimension_semantics=("parallel", "parallel", "arbitrary")),
    )(chosen, q, k, v)


# speedup vs baseline: 1.0211x; 1.0178x over previous
import functools

import jax
import jax.numpy as jnp
import numpy as np
from jax.experimental import pallas as pl
import jax.experimental.pallas.tpu as pltpu
from jax.experimental.pallas import tpu_sc as plsc

FRAG = 64
HALF = FRAG // 2
G = 6
P = 24
PPAD = 32
GPAD = 8
KQ = 5
TOP_U = 16
W_MEAN, W_MAX, W_TOPK, W_STD = 1.0, 0.6, 0.4, 0.2
ALPHA = 0.15
TAU = 8.0
NEG = -1e30


def _stats_kernel(q_ref, k_ref, u_ref, st_ref, *, T, d):
    kh = k_ref[0]
    qh = q_ref[0]

    r32 = jax.lax.broadcasted_iota(jnp.int32, (PPAD, 1), 0)
    step = float(T - 1) / float(P - 1)
    idxp = jnp.where(r32 < P,
                     jnp.round(r32.astype(jnp.float32) * step).astype(jnp.int32),
                     -1)
    colT = jax.lax.broadcasted_iota(jnp.int32, (PPAD, T), 1)
    onehot_p = (colT == idxp).astype(jnp.float32)
    qp = jax.lax.dot_general(onehot_p, qh, (((1,), (0,)), ((), ())),
                             preferred_element_type=jnp.float32,
                             precision=jax.lax.Precision.HIGHEST)
    qnorm = jnp.sqrt(jnp.sum(qp * qp, axis=1, keepdims=True))
    qp = qp / jnp.maximum(qnorm, 1e-6)

    knorm = jnp.sqrt(jnp.sum(kh * kh, axis=1, keepdims=True))
    kbar = kh / jnp.maximum(knorm, 1e-6)

    st = jax.lax.dot_general(qp.astype(jnp.bfloat16), kbar.astype(jnp.bfloat16),
                             (((1,), (1,)), ((), ())),
                             preferred_element_type=jnp.float32)
    st = jnp.maximum(st, 0.0)

    mean = jnp.sum(st, axis=0, keepdims=True) / P
    mx = jnp.max(st, axis=0, keepdims=True)
    ssq = jnp.sum(st * st, axis=0, keepdims=True)
    std = jnp.sqrt(jnp.maximum(ssq / P - mean * mean, 0.0))

    sub = jax.lax.broadcasted_iota(jnp.int32, (PPAD, T), 0)
    cur = st
    acc = jnp.zeros((1, T), jnp.float32)
    for _ in range(KQ):
        m5 = jnp.max(cur, axis=0, keepdims=True)
        acc = acc + m5
        first = jnp.min(jnp.where(cur >= m5, sub, PPAD), axis=0, keepdims=True)
        cur = jnp.where(sub == first, -1.0, cur)
    topk_mean = acc / KQ

    u = W_MEAN * mean + W_MAX * mx + W_TOPK * topk_mean + W_STD * std
    u_ref[0] = u
    st_ref[0] = st[:P]


def _router_sc_body(u_hbm, s_hbm, out_hbm, u_v, s_v, cho_v, *, T):
    c = jax.lax.axis_index("c")
    s = jax.lax.axis_index("s")
    h = s

    @pl.when(c == 0)
    def _work():
        pltpu.sync_copy(u_hbm.at[h], u_v)
        pltpu.sync_copy(s_hbm.at[h], s_v)
        iota16 = jax.lax.iota(jnp.int32, 16)
        neg16 = jnp.full((16,), -3e38, jnp.float32)
        cand = jnp.zeros((16,), jnp.int32)
        nch = T // 16

        for r in range(TOP_U):
            def scan_body(ci, carry):
                bv, bi = carry
                chunk = u_v[pl.ds(ci * 16, 16)]
                m = jnp.max(chunk)
                fl = jnp.max(plsc.all_reduce_ffs(chunk == m))
                idx = ci * 16 + fl
                better = m > bv
                return (jnp.where(better, m, bv),
                        jnp.where(better, idx, bi))
            bv, bi = jax.lax.fori_loop(
                0, nch, scan_body,
                (jnp.float32(-3e38), jnp.int32(0)))
            c0 = bi // 16
            l0 = bi - c0 * 16
            ch = u_v[pl.ds(c0 * 16, 16)]
            u_v[pl.ds(c0 * 16, 16)] = jnp.where(iota16 == l0, neg16, ch)
            cand = jnp.where(iota16 == r, bi, cand)

        ssub = []
        for p in range(P):
            ssub.append(plsc.load_gather(
                s_v, [jnp.full((16,), p, jnp.int32), cand]))

        msub = [jnp.float32(0.0)] * P
        blocked = jnp.zeros((16,), jnp.bool_)
        chosen = jnp.zeros((16,), jnp.int32)
        for r in range(G):
            gains = jnp.zeros((16,), jnp.float32)
            for p in range(P):
                gains = gains + jnp.maximum(ssub[p] - msub[p], 0.0)
            gains = jnp.where(blocked, jnp.float32(-1e9), gains)
            mg = jnp.max(gains)
            j = jnp.max(plsc.all_reduce_ffs(gains == mg))
            ohj = iota16 == j
            blocked = blocked | ohj
            cj = jnp.sum(jnp.where(ohj, cand, 0))
            chosen = jnp.where(iota16 == r, cj, chosen)
            for p in range(P):
                sp = jnp.sum(jnp.where(ohj, ssub[p], jnp.float32(0.0)))
                msub[p] = jnp.maximum(msub[p], sp)
        cho_v[...] = chosen
        pltpu.sync_copy(cho_v, out_hbm.at[h])


def _attn_kernel(chosen_ref, q_ref, k_ref, v_ref, o_ref, *, T, d, C, HALO):
    h = pl.program_id(0)
    b = pl.program_id(1)
    s0 = b * C
    halo_start = jnp.clip(s0 - HALF, 0, T - HALO)

    qb = q_ref[0]
    k_halo = k_ref[0, pl.ds(halo_start, HALO), :]
    v_halo = v_ref[0, pl.ds(halo_start, HALO), :]

    inv_sqrt_d = 1.0 / float(np.sqrt(d))
    qb16 = qb.astype(jnp.bfloat16)
    sl = jax.lax.dot_general(qb16, k_halo.astype(jnp.bfloat16),
                             (((1,), (1,)), ((), ())),
                             preferred_element_type=jnp.float32) * inv_sqrt_d
    t_abs = s0 + jax.lax.broadcasted_iota(jnp.int32, (C, HALO), 0)
    j_abs = halo_start + jax.lax.broadcasted_iota(jnp.int32, (C, HALO), 1)
    start_t = jnp.clip(t_abs - HALF, 0, T - FRAG)
    valid = (j_abs >= start_t) & (j_abs < start_t + FRAG)
    prior = jnp.exp(jnp.abs(j_abs - t_abs).astype(jnp.float32) * (-1.0 / TAU))
    sl = jnp.where(valid, sl + ALPHA * prior, NEG)

    krows = []
    vrows = []
    cvals = []
    for i in range(G):
        ci = chosen_ref[h, i]
        krows.append(k_ref[0, pl.ds(ci, 1), :])
        vrows.append(v_ref[0, pl.ds(ci, 1), :])
        cvals.append(ci.reshape(1, 1))
    zpad = jnp.zeros((GPAD - G, d), jnp.float32)
    kg = jnp.concatenate(krows + [zpad], axis=0)
    vg = jnp.concatenate(vrows + [zpad], axis=0)
    crow = jnp.concatenate(
        cvals + [jnp.zeros((1, GPAD - G), jnp.int32)], axis=1)

    sg = jax.lax.dot_general(qb16, kg.astype(jnp.bfloat16),
                             (((1,), (1,)), ((), ())),
                             preferred_element_type=jnp.float32) * inv_sqrt_d
    tq = s0 + jax.lax.broadcasted_iota(jnp.int32, (C, GPAD), 0)
    lane8 = jax.lax.broadcasted_iota(jnp.int32, (C, GPAD), 1)
    priorg = jnp.exp(jnp.abs(crow - tq).astype(jnp.float32) * (-1.0 / TAU))
    sg = jnp.where(lane8 < G, sg + ALPHA * priorg, NEG)

    mm = jnp.maximum(jnp.max(sl, axis=1, keepdims=True),
                     jnp.max(sg, axis=1, keepdims=True))
    pl_ = jnp.exp(sl - mm)
    pg = jnp.exp(sg - mm)
    denom = jnp.sum(pl_, axis=1, keepdims=True) + jnp.sum(pg, axis=1, keepdims=True)
    ctx = (jnp.dot(pl_.astype(jnp.bfloat16), v_halo.astype(jnp.bfloat16),
                   preferred_element_type=jnp.float32)
           + jnp.dot(pg.astype(jnp.bfloat16), vg.astype(jnp.bfloat16),
                     preferred_element_type=jnp.float32)) / denom
    o_ref[0] = ctx


@jax.jit
def kernel(q, k, v):
    B, H, T, d = q.shape
    BH = B * H
    qf = q.reshape(BH, T, d)
    kf = k.reshape(BH, T, d)
    vf = v.reshape(BH, T, d)

    u3, st3 = pl.pallas_call(
        functools.partial(_stats_kernel, T=T, d=d),
        grid=(BH,),
        in_specs=[
            pl.BlockSpec((1, T, d), lambda h: (h, 0, 0)),
            pl.BlockSpec((1, T, d), lambda h: (h, 0, 0)),
        ],
        out_specs=[
            pl.BlockSpec((1, 1, T), lambda h: (h, 0, 0)),
            pl.BlockSpec((1, P, T), lambda h: (h, 0, 0)),
        ],
        out_shape=[
            jax.ShapeDtypeStruct((BH, 1, T), jnp.float32),
            jax.ShapeDtypeStruct((BH, P, T), jnp.float32),
        ],
        compiler_params=pltpu.CompilerParams(
            dimension_semantics=("parallel",)),
    )(qf, kf)

    mesh = plsc.VectorSubcoreMesh(core_axis_name="c", subcore_axis_name="s")
    chosen = pl.kernel(
        functools.partial(_router_sc_body, T=T),
        out_type=jax.ShapeDtypeStruct((BH, 16), jnp.int32),
        mesh=mesh,
        scratch_types=[
            pltpu.VMEM((T,), jnp.float32),
            pltpu.VMEM((P, T), jnp.float32),
            pltpu.VMEM((16,), jnp.int32),
        ],
        compiler_params=pltpu.CompilerParams(needs_layout_passes=False),
    )(u3.reshape(BH, T), st3)

    C = 256
    HALO = C + FRAG
    NB = T // C
    grid_spec = pltpu.PrefetchScalarGridSpec(
        num_scalar_prefetch=1,
        grid=(BH, NB),
        in_specs=[
            pl.BlockSpec((1, C, d), lambda h, b, ch: (h, b, 0)),
            pl.BlockSpec((1, T, d), lambda h, b, ch: (h, 0, 0)),
            pl.BlockSpec((1, T, d), lambda h, b, ch: (h, 0, 0)),
        ],
        out_specs=pl.BlockSpec((1, C, d), lambda h, b, ch: (h, b, 0)),
    )
    ctx = pl.pallas_call(
        functools.partial(_attn_kernel, T=T, d=d, C=C, HALO=HALO),
        grid_spec=grid_spec,
        out_shape=jax.ShapeDtypeStruct((BH, T, d), jnp.float32),
        compiler_params=pltpu.CompilerParams(
            dimension_semantics=("parallel", "arbitrary")),
    )(chosen, qf, kf, vf)
    return ctx.reshape(B, H, T, d)
